# initial kernel scaffold (unmeasured)
import jax
import jax.numpy as jnp
from jax import lax
from jax.experimental import pallas as pl
from jax.experimental.pallas import tpu as pltpu

N_DEV = 4
N_EXP = 16
E_LOC = N_EXP // N_DEV
CAP = 204


def kernel(x, router_W, route_idx, expert_W):
    del router_W
    m, d = x.shape
    _, _, h = expert_W.shape

    def body(x_ref, route_ref, w_ref, out_ref,
             w_all, route_all, w_send, w_recv, r_send, r_recv):
        s = lax.axis_index("i")
        left = lax.rem(s - 1 + N_DEV, N_DEV)
        right = lax.rem(s + 1, N_DEV)

        barrier = pltpu.get_barrier_semaphore()
        for nbr in (left, right):
            pl.semaphore_signal(barrier, inc=1, device_id=(nbr,),
                                device_id_type=pl.DeviceIdType.MESH)
        pl.semaphore_wait(barrier, 2)

        w_all[pl.ds(s * E_LOC, E_LOC)] = w_ref[...].astype(jnp.bfloat16)
        route_all[pl.ds(s, 1)] = route_ref[...][None]

        for k in range(N_DEV - 1):
            o_send = lax.rem(s - k + N_DEV, N_DEV)
            rdma_w = pltpu.make_async_remote_copy(
                src_ref=w_all.at[pl.ds(o_send * E_LOC, E_LOC)],
                dst_ref=w_all.at[pl.ds(o_send * E_LOC, E_LOC)],
                send_sem=w_send.at[k],
                recv_sem=w_recv.at[k],
                device_id=(right,),
                device_id_type=pl.DeviceIdType.MESH,
            )
            rdma_r = pltpu.make_async_remote_copy(
                src_ref=route_all.at[pl.ds(o_send, 1)],
                dst_ref=route_all.at[pl.ds(o_send, 1)],
                send_sem=r_send.at[k],
                recv_sem=r_recv.at[k],
                device_id=(right,),
                device_id_type=pl.DeviceIdType.MESH,
            )
            rdma_w.start()
            rdma_r.start()
            rdma_w.wait()
            rdma_r.wait()

        eids = lax.broadcasted_iota(jnp.int32, (m, N_EXP), 1)
        route_own = route_ref[...]
        R = (route_own == eids)
        R_f = R.astype(jnp.float32)
        R_b = R.astype(jnp.bfloat16)

        offs = jnp.zeros((1, N_EXP), jnp.float32)
        for sp in range(N_DEV):
            R_sp = (route_all[sp] == eids).astype(jnp.float32)
            cnt = jnp.sum(R_sp, axis=0, keepdims=True)
            offs = offs + jnp.where(sp < s, cnt, 0.0)

        row = lax.broadcasted_iota(jnp.int32, (m, m), 0)
        col = lax.broadcasted_iota(jnp.int32, (m, m), 1)
        L = (col < row).astype(jnp.bfloat16)
        cum = jnp.dot(L, R_b, preferred_element_type=jnp.float32)
        rank = jnp.sum(cum * R_f, axis=1, keepdims=True)
        off_tok = jnp.sum(offs * R_f, axis=1, keepdims=True)
        keep = ((rank + off_tok) < float(CAP)).astype(jnp.bfloat16)

        xb = x_ref[...].astype(jnp.bfloat16)
        acc = jnp.zeros((m, h), jnp.float32)
        for e in range(N_EXP):
            m_e = keep * R_b[:, e:e + 1]
            acc = acc + jnp.dot(xb * m_e, w_all[e],
                                preferred_element_type=jnp.float32)
        out_ref[...] = acc

    return pl.pallas_call(
        body,
        out_shape=jax.ShapeDtypeStruct((m, h), jnp.float32),
        in_specs=[pl.BlockSpec(memory_space=pltpu.VMEM)] * 3,
        out_specs=pl.BlockSpec(memory_space=pltpu.VMEM),
        scratch_shapes=[
            pltpu.VMEM((N_EXP, d, h), jnp.bfloat16),
            pltpu.VMEM((N_DEV, m, 1), jnp.int32),
            pltpu.SemaphoreType.DMA((N_DEV - 1,)),
            pltpu.SemaphoreType.DMA((N_DEV - 1,)),
            pltpu.SemaphoreType.DMA((N_DEV - 1,)),
            pltpu.SemaphoreType.DMA((N_DEV - 1,)),
        ],
        compiler_params=pltpu.CompilerParams(collective_id=0),
    )(x, route_idx, expert_W)


# baseline (device time: 192426 ns/iter reference)
import jax
import jax.numpy as jnp
from jax import lax
from jax.experimental import pallas as pl
from jax.experimental.pallas import tpu as pltpu

N_DEV = 4
N_EXP = 16
E_LOC = N_EXP // N_DEV
CAP = 204


def kernel(x, router_W, route_idx, expert_W):
    del router_W
    m, d = x.shape
    _, _, h = expert_W.shape

    def body(x_ref, route_ref, w_ref, out_ref,
             w_all, route_all, w_send, w_recv, r_send, r_recv):
        s = lax.axis_index("i")
        left = lax.rem(s - 1 + N_DEV, N_DEV)
        right = lax.rem(s + 1, N_DEV)

        barrier = pltpu.get_barrier_semaphore()
        for nbr in (left, right):
            pl.semaphore_signal(barrier, inc=1, device_id=(nbr,),
                                device_id_type=pl.DeviceIdType.MESH)
        pl.semaphore_wait(barrier, 2)

        w_all[pl.ds(s * E_LOC, E_LOC)] = w_ref[...].astype(jnp.bfloat16)
        route_all[pl.ds(s, 1)] = route_ref[...][None]

        for k in range(N_DEV - 1):
            o_send = lax.rem(s - k + N_DEV, N_DEV)
            rdma_w = pltpu.make_async_remote_copy(
                src_ref=w_all.at[pl.ds(o_send * E_LOC, E_LOC)],
                dst_ref=w_all.at[pl.ds(o_send * E_LOC, E_LOC)],
                send_sem=w_send.at[k],
                recv_sem=w_recv.at[k],
                device_id=(right,),
                device_id_type=pl.DeviceIdType.MESH,
            )
            rdma_r = pltpu.make_async_remote_copy(
                src_ref=route_all.at[pl.ds(o_send, 1)],
                dst_ref=route_all.at[pl.ds(o_send, 1)],
                send_sem=r_send.at[k],
                recv_sem=r_recv.at[k],
                device_id=(right,),
                device_id_type=pl.DeviceIdType.MESH,
            )
            rdma_w.start()
            rdma_r.start()
            rdma_w.wait()
            rdma_r.wait()

        eids = lax.broadcasted_iota(jnp.int32, (m, N_EXP), 1)
        route_own = route_ref[...]
        R = (route_own == eids)
        R_f = R.astype(jnp.float32)
        R_b = R.astype(jnp.bfloat16)

        offs = jnp.zeros((1, N_EXP), jnp.float32)
        for sp in range(N_DEV):
            R_sp = (route_all[sp] == eids).astype(jnp.float32)
            cnt = jnp.sum(R_sp, axis=0, keepdims=True)
            offs = offs + jnp.where(sp < s, cnt, 0.0)

        row = lax.broadcasted_iota(jnp.int32, (m, m), 0)
        col = lax.broadcasted_iota(jnp.int32, (m, m), 1)
        L = (col < row).astype(jnp.bfloat16)
        cum = jnp.dot(L, R_b, preferred_element_type=jnp.float32)
        rank = jnp.sum(cum * R_f, axis=1, keepdims=True)
        off_tok = jnp.sum(offs * R_f, axis=1, keepdims=True)
        keep = ((rank + off_tok) < float(CAP)).astype(jnp.bfloat16)

        xb = x_ref[...].astype(jnp.bfloat16)
        acc = jnp.zeros((m, h), jnp.float32)
        for e in range(N_EXP):
            m_e = keep * R_b[:, e:e + 1]
            acc = acc + jnp.dot(xb * m_e, w_all[e],
                                preferred_element_type=jnp.float32)
        out_ref[...] = acc

    return pl.pallas_call(
        body,
        out_shape=jax.ShapeDtypeStruct((m, h), jnp.float32),
        in_specs=[pl.BlockSpec(memory_space=pltpu.VMEM)] * 3,
        out_specs=pl.BlockSpec(memory_space=pltpu.VMEM),
        scratch_shapes=[
            pltpu.VMEM((N_EXP, d, h), jnp.bfloat16),
            pltpu.VMEM((N_DEV, m, 1), jnp.int32),
            pltpu.SemaphoreType.DMA((N_DEV - 1,)),
            pltpu.SemaphoreType.DMA((N_DEV - 1,)),
            pltpu.SemaphoreType.DMA((N_DEV - 1,)),
            pltpu.SemaphoreType.DMA((N_DEV - 1,)),
        ],
        compiler_params=pltpu.CompilerParams(
            collective_id=0, vmem_limit_bytes=100 * 1024 * 1024
        ),
    )(x, route_idx, expert_W)


# device time: 118025 ns/iter; 1.6304x vs baseline; 1.6304x over previous
import jax
import jax.numpy as jnp
from jax import lax
from jax.experimental import pallas as pl
from jax.experimental.pallas import tpu as pltpu

N_DEV = 4
N_EXP = 16
E_LOC = N_EXP // N_DEV
HALF = E_LOC // 2
CAP = 204


def kernel(x, router_W, route_idx, expert_W):
    del router_W
    m, d = x.shape
    _, _, h = expert_W.shape

    def body(x_ref, route_ref, w_ref, out_ref,
             w_all, route_all, xb_ref,
             wr_send, wr_recv, wl_send, wl_recv, r_send, r_recv):
        s = lax.axis_index("i")
        left = lax.rem(s - 1 + N_DEV, N_DEV)
        right = lax.rem(s + 1, N_DEV)

        barrier = pltpu.get_barrier_semaphore()
        for dd in (1, 2, 3):
            peer = lax.rem(s + dd, N_DEV)
            pl.semaphore_signal(barrier, inc=1, device_id=(peer,),
                                device_id_type=pl.DeviceIdType.MESH)
        pl.semaphore_wait(barrier, N_DEV - 1)

        w_all[pl.ds(s * E_LOC, E_LOC)] = w_ref[...].astype(jnp.bfloat16)
        route_all[pl.ds(s, 1)] = route_ref[...][None]

        def w_slice(origin, half):
            return pl.ds(origin * E_LOC + half * HALF, HALF)

        sR, rR = [], []
        for k in range(N_DEV - 1):
            o_s = lax.rem(s - k + N_DEV, N_DEV)
            sR.append(pltpu.make_async_remote_copy(
                src_ref=w_all.at[w_slice(o_s, 0)],
                dst_ref=w_all.at[w_slice(o_s, 0)],
                send_sem=wr_send.at[k], recv_sem=wr_recv.at[k],
                device_id=(right,), device_id_type=pl.DeviceIdType.MESH,
            ))
            o_r = lax.rem(s - k - 1 + N_DEV, N_DEV)
            rR.append(pltpu.make_async_remote_copy(
                src_ref=w_all.at[w_slice(o_r, 0)],
                dst_ref=w_all.at[w_slice(o_r, 0)],
                send_sem=wr_send.at[k], recv_sem=wr_recv.at[k],
                device_id=(left,), device_id_type=pl.DeviceIdType.MESH,
            ))
        sL, rL = [], []
        for k in range(N_DEV - 1):
            o_s = lax.rem(s + k, N_DEV)
            sL.append(pltpu.make_async_remote_copy(
                src_ref=w_all.at[w_slice(o_s, 1)],
                dst_ref=w_all.at[w_slice(o_s, 1)],
                send_sem=wl_send.at[k], recv_sem=wl_recv.at[k],
                device_id=(left,), device_id_type=pl.DeviceIdType.MESH,
            ))
            o_r = lax.rem(s + k + 1, N_DEV)
            rL.append(pltpu.make_async_remote_copy(
                src_ref=w_all.at[w_slice(o_r, 1)],
                dst_ref=w_all.at[w_slice(o_r, 1)],
                send_sem=wl_send.at[k], recv_sem=wl_recv.at[k],
                device_id=(right,), device_id_type=pl.DeviceIdType.MESH,
            ))

        sR[0].start()
        sL[0].start()
        route_sends = []
        for dd in (1, 2, 3):
            peer = lax.rem(s + dd, N_DEV)
            rs = pltpu.make_async_remote_copy(
                src_ref=route_all.at[pl.ds(s, 1)],
                dst_ref=route_all.at[pl.ds(s, 1)],
                send_sem=r_send.at[dd - 1], recv_sem=r_recv.at[s],
                device_id=(peer,), device_id_type=pl.DeviceIdType.MESH,
            )
            rs.start()
            route_sends.append(rs)

        for dd in (1, 2, 3):
            o = lax.rem(s + dd, N_DEV)
            pltpu.make_async_remote_copy(
                src_ref=route_all.at[pl.ds(o, 1)],
                dst_ref=route_all.at[pl.ds(o, 1)],
                send_sem=r_send.at[dd - 1], recv_sem=r_recv.at[o],
                device_id=(s,), device_id_type=pl.DeviceIdType.MESH,
            ).wait_recv()

        eids = lax.broadcasted_iota(jnp.int32, (m, N_EXP), 1)
        route_own = route_ref[...]
        R = (route_own == eids)
        R_f = R.astype(jnp.float32)
        R_b = R.astype(jnp.bfloat16)

        offs = jnp.zeros((1, N_EXP), jnp.float32)
        for sp in range(N_DEV):
            R_sp = (route_all[sp] == eids).astype(jnp.float32)
            cnt = jnp.sum(R_sp, axis=0, keepdims=True)
            offs = offs + jnp.where(sp < s, cnt, 0.0)

        row = lax.broadcasted_iota(jnp.int32, (m, m), 0)
        col = lax.broadcasted_iota(jnp.int32, (m, m), 1)
        L = (col < row).astype(jnp.bfloat16)
        cum = jnp.dot(L, R_b, preferred_element_type=jnp.float32)
        rank = jnp.sum(cum * R_f, axis=1, keepdims=True)
        off_tok = jnp.sum(offs * R_f, axis=1, keepdims=True)
        keep = ((rank + off_tok) < float(CAP)).astype(jnp.bfloat16)

        xb_ref[...] = x_ref[...].astype(jnp.bfloat16)
        out_ref[...] = jnp.zeros((m, h), jnp.float32)

        def expert_mm(e):
            msk = keep * (route_own == e).astype(jnp.bfloat16)
            w_e = w_all[pl.ds(e, 1)][0]
            out_ref[...] = out_ref[...] + jnp.dot(
                xb_ref[...] * msk, w_e, preferred_element_type=jnp.float32)

        for j in range(E_LOC):
            expert_mm(s * E_LOC + j)

        for k in range(N_DEV - 1):
            rR[k].wait_recv()
            if k + 1 < N_DEV - 1:
                sR[k + 1].start()
            o = lax.rem(s - k - 1 + N_DEV, N_DEV)
            for j in range(HALF):
                expert_mm(o * E_LOC + j)

            rL[k].wait_recv()
            if k + 1 < N_DEV - 1:
                sL[k + 1].start()
            o = lax.rem(s + k + 1, N_DEV)
            for j in range(HALF):
                expert_mm(o * E_LOC + HALF + j)

        for k in range(N_DEV - 1):
            sR[k].wait_send()
            sL[k].wait_send()
        for rs in route_sends:
            rs.wait_send()

    return pl.pallas_call(
        body,
        out_shape=jax.ShapeDtypeStruct((m, h), jnp.float32),
        in_specs=[pl.BlockSpec(memory_space=pltpu.VMEM)] * 3,
        out_specs=pl.BlockSpec(memory_space=pltpu.VMEM),
        scratch_shapes=[
            pltpu.VMEM((N_EXP, d, h), jnp.bfloat16),
            pltpu.VMEM((N_DEV, m, 1), jnp.int32),
            pltpu.VMEM((m, d), jnp.bfloat16),
            pltpu.SemaphoreType.DMA((N_DEV - 1,)),
            pltpu.SemaphoreType.DMA((N_DEV - 1,)),
            pltpu.SemaphoreType.DMA((N_DEV - 1,)),
            pltpu.SemaphoreType.DMA((N_DEV - 1,)),
            pltpu.SemaphoreType.DMA((N_DEV - 1,)),
            pltpu.SemaphoreType.DMA((N_DEV,)),
        ],
        compiler_params=pltpu.CompilerParams(
            collective_id=0, vmem_limit_bytes=100 * 1024 * 1024
        ),
    )(x, route_idx, expert_W)


# device time: 117752 ns/iter; 1.6342x vs baseline; 1.0023x over previous
import jax
import jax.numpy as jnp
from jax import lax
from jax.experimental import pallas as pl
from jax.experimental.pallas import tpu as pltpu

N_DEV = 4
N_EXP = 16
E_LOC = N_EXP // N_DEV
HALF = E_LOC // 2
CAP = 204


def kernel(x, router_W, route_idx, expert_W):
    del router_W
    m, d = x.shape
    _, _, h = expert_W.shape

    def body(x_ref, route_ref, w_ref, out_ref,
             w_all, route_all, xb_ref,
             wr_send, wr_recv, wl_send, wl_recv, r_send, r_recv):
        s = lax.axis_index("i")
        left = lax.rem(s - 1 + N_DEV, N_DEV)
        right = lax.rem(s + 1, N_DEV)

        barrier = pltpu.get_barrier_semaphore()
        for dd in (1, 2, 3):
            peer = lax.rem(s + dd, N_DEV)
            pl.semaphore_signal(barrier, inc=1, device_id=(peer,),
                                device_id_type=pl.DeviceIdType.MESH)
        pl.semaphore_wait(barrier, N_DEV - 1)

        def w_slice(origin, half):
            return pl.ds(origin * E_LOC + half * HALF, HALF)

        sR, rR = [], []
        for k in range(N_DEV - 1):
            o_s = lax.rem(s - k + N_DEV, N_DEV)
            sR.append(pltpu.make_async_remote_copy(
                src_ref=w_all.at[w_slice(o_s, 0)],
                dst_ref=w_all.at[w_slice(o_s, 0)],
                send_sem=wr_send.at[k], recv_sem=wr_recv.at[k],
                device_id=(right,), device_id_type=pl.DeviceIdType.MESH,
            ))
            o_r = lax.rem(s - k - 1 + N_DEV, N_DEV)
            rR.append(pltpu.make_async_remote_copy(
                src_ref=w_all.at[w_slice(o_r, 0)],
                dst_ref=w_all.at[w_slice(o_r, 0)],
                send_sem=wr_send.at[k], recv_sem=wr_recv.at[k],
                device_id=(left,), device_id_type=pl.DeviceIdType.MESH,
            ))
        sL, rL = [], []
        for k in range(N_DEV - 1):
            o_s = lax.rem(s + k, N_DEV)
            sL.append(pltpu.make_async_remote_copy(
                src_ref=w_all.at[w_slice(o_s, 1)],
                dst_ref=w_all.at[w_slice(o_s, 1)],
                send_sem=wl_send.at[k], recv_sem=wl_recv.at[k],
                device_id=(left,), device_id_type=pl.DeviceIdType.MESH,
            ))
            o_r = lax.rem(s + k + 1, N_DEV)
            rL.append(pltpu.make_async_remote_copy(
                src_ref=w_all.at[w_slice(o_r, 1)],
                dst_ref=w_all.at[w_slice(o_r, 1)],
                send_sem=wl_send.at[k], recv_sem=wl_recv.at[k],
                device_id=(right,), device_id_type=pl.DeviceIdType.MESH,
            ))

        w_all[w_slice(s, 0)] = w_ref[pl.ds(0, HALF)].astype(jnp.bfloat16)
        sR[0].start()
        w_all[w_slice(s, 1)] = w_ref[pl.ds(HALF, HALF)].astype(jnp.bfloat16)
        sL[0].start()
        route_all[pl.ds(s, 1)] = route_ref[...][None]
        route_sends = []
        for dd in (1, 2, 3):
            peer = lax.rem(s + dd, N_DEV)
            rs = pltpu.make_async_remote_copy(
                src_ref=route_all.at[pl.ds(s, 1)],
                dst_ref=route_all.at[pl.ds(s, 1)],
                send_sem=r_send.at[dd - 1], recv_sem=r_recv.at[s],
                device_id=(peer,), device_id_type=pl.DeviceIdType.MESH,
            )
            rs.start()
            route_sends.append(rs)

        for dd in (1, 2, 3):
            o = lax.rem(s + dd, N_DEV)
            pltpu.make_async_remote_copy(
                src_ref=route_all.at[pl.ds(o, 1)],
                dst_ref=route_all.at[pl.ds(o, 1)],
                send_sem=r_send.at[dd - 1], recv_sem=r_recv.at[o],
                device_id=(s,), device_id_type=pl.DeviceIdType.MESH,
            ).wait_recv()

        eids = lax.broadcasted_iota(jnp.int32, (m, N_EXP), 1)
        route_own = route_ref[...]
        R = (route_own == eids)
        R_f = R.astype(jnp.float32)
        R_b = R.astype(jnp.bfloat16)

        offs = jnp.zeros((1, N_EXP), jnp.float32)
        for sp in range(N_DEV):
            R_sp = (route_all[sp] == eids).astype(jnp.float32)
            cnt = jnp.sum(R_sp, axis=0, keepdims=True)
            offs = offs + jnp.where(sp < s, cnt, 0.0)

        row = lax.broadcasted_iota(jnp.int32, (m, m), 0)
        col = lax.broadcasted_iota(jnp.int32, (m, m), 1)
        L = (col < row).astype(jnp.bfloat16)
        cum = jnp.dot(L, R_b, preferred_element_type=jnp.float32)
        rank = jnp.sum(cum * R_f, axis=1, keepdims=True)
        off_tok = jnp.sum(offs * R_f, axis=1, keepdims=True)
        keep = ((rank + off_tok) < float(CAP)).astype(jnp.bfloat16)

        xb_ref[...] = x_ref[...].astype(jnp.bfloat16)

        def expert_mm(e, init=False):
            msk = keep * (route_own == e).astype(jnp.bfloat16)
            w_e = w_all[pl.ds(e, 1)][0]
            mm = jnp.dot(xb_ref[...] * msk, w_e,
                         preferred_element_type=jnp.float32)
            out_ref[...] = mm if init else out_ref[...] + mm

        for j in range(E_LOC):
            expert_mm(s * E_LOC + j, init=(j == 0))

        for k in range(N_DEV - 1):
            rR[k].wait_recv()
            rL[k].wait_recv()
            if k + 1 < N_DEV - 1:
                sR[k + 1].start()
                sL[k + 1].start()
            oR = lax.rem(s - k - 1 + N_DEV, N_DEV)
            oL = lax.rem(s + k + 1, N_DEV)
            for j in range(HALF):
                expert_mm(oR * E_LOC + j)
            for j in range(HALF):
                expert_mm(oL * E_LOC + HALF + j)

        for k in range(N_DEV - 1):
            sR[k].wait_send()
            sL[k].wait_send()
        for rs in route_sends:
            rs.wait_send()

    return pl.pallas_call(
        body,
        out_shape=jax.ShapeDtypeStruct((m, h), jnp.float32),
        in_specs=[pl.BlockSpec(memory_space=pltpu.VMEM)] * 3,
        out_specs=pl.BlockSpec(memory_space=pltpu.VMEM),
        scratch_shapes=[
            pltpu.VMEM((N_EXP, d, h), jnp.bfloat16),
            pltpu.VMEM((N_DEV, m, 1), jnp.int32),
            pltpu.VMEM((m, d), jnp.bfloat16),
            pltpu.SemaphoreType.DMA((N_DEV - 1,)),
            pltpu.SemaphoreType.DMA((N_DEV - 1,)),
            pltpu.SemaphoreType.DMA((N_DEV - 1,)),
            pltpu.SemaphoreType.DMA((N_DEV - 1,)),
            pltpu.SemaphoreType.DMA((N_DEV - 1,)),
            pltpu.SemaphoreType.DMA((N_DEV,)),
        ],
        compiler_params=pltpu.CompilerParams(
            collective_id=0, vmem_limit_bytes=100 * 1024 * 1024
        ),
    )(x, route_idx, expert_W)


# device time: 106719 ns/iter; 1.8031x vs baseline; 1.1034x over previous
import jax
import jax.numpy as jnp
from jax import lax
from jax.experimental import pallas as pl
from jax.experimental.pallas import tpu as pltpu

N_DEV = 4
N_EXP = 16
E_LOC = N_EXP // N_DEV
HALF = E_LOC // 2
CAP = 204


def kernel(x, router_W, route_idx, expert_W):
    del router_W
    m, d = x.shape
    _, _, h = expert_W.shape

    def body(x_ref, route_ref, w_ref, out_ref,
             w_all, route_all, xb_ref,
             wr_send, wr_recv, wl_send, wl_recv, r_send, r_recv):
        s = lax.axis_index("i")
        left = lax.rem(s - 1 + N_DEV, N_DEV)
        right = lax.rem(s + 1, N_DEV)

        barrier = pltpu.get_barrier_semaphore()
        for dd in (1, 2, 3):
            peer = lax.rem(s + dd, N_DEV)
            pl.semaphore_signal(barrier, inc=1, device_id=(peer,),
                                device_id_type=pl.DeviceIdType.MESH)
        pl.semaphore_wait(barrier, N_DEV - 1)

        def w_slice(origin, half):
            return pl.ds(origin * E_LOC + half * HALF, HALF)

        sR, rR = [], []
        for k in range(N_DEV - 1):
            o_s = lax.rem(s - k + N_DEV, N_DEV)
            sR.append(pltpu.make_async_remote_copy(
                src_ref=w_all.at[w_slice(o_s, 0)],
                dst_ref=w_all.at[w_slice(o_s, 0)],
                send_sem=wr_send.at[k], recv_sem=wr_recv.at[k],
                device_id=(right,), device_id_type=pl.DeviceIdType.MESH,
            ))
            o_r = lax.rem(s - k - 1 + N_DEV, N_DEV)
            rR.append(pltpu.make_async_remote_copy(
                src_ref=w_all.at[w_slice(o_r, 0)],
                dst_ref=w_all.at[w_slice(o_r, 0)],
                send_sem=wr_send.at[k], recv_sem=wr_recv.at[k],
                device_id=(left,), device_id_type=pl.DeviceIdType.MESH,
            ))
        sL, rL = [], []
        for k in range(N_DEV - 1):
            o_s = lax.rem(s + k, N_DEV)
            sL.append(pltpu.make_async_remote_copy(
                src_ref=w_all.at[w_slice(o_s, 1)],
                dst_ref=w_all.at[w_slice(o_s, 1)],
                send_sem=wl_send.at[k], recv_sem=wl_recv.at[k],
                device_id=(left,), device_id_type=pl.DeviceIdType.MESH,
            ))
            o_r = lax.rem(s + k + 1, N_DEV)
            rL.append(pltpu.make_async_remote_copy(
                src_ref=w_all.at[w_slice(o_r, 1)],
                dst_ref=w_all.at[w_slice(o_r, 1)],
                send_sem=wl_send.at[k], recv_sem=wl_recv.at[k],
                device_id=(right,), device_id_type=pl.DeviceIdType.MESH,
            ))

        route_all[pl.ds(s, 1)] = route_ref[...][None]
        route_sends = []
        for dd in (1, 2, 3):
            peer = lax.rem(s + dd, N_DEV)
            rs = pltpu.make_async_remote_copy(
                src_ref=route_all.at[pl.ds(s, 1)],
                dst_ref=route_all.at[pl.ds(s, 1)],
                send_sem=r_send.at[dd - 1], recv_sem=r_recv.at[s],
                device_id=(peer,), device_id_type=pl.DeviceIdType.MESH,
            )
            rs.start()
            route_sends.append(rs)

        w_all[w_slice(s, 0)] = w_ref[pl.ds(0, HALF)].astype(jnp.bfloat16)
        sR[0].start()
        w_all[w_slice(s, 1)] = w_ref[pl.ds(HALF, HALF)].astype(jnp.bfloat16)
        sL[0].start()

        for dd in (1, 2, 3):
            o = lax.rem(s + dd, N_DEV)
            pltpu.make_async_remote_copy(
                src_ref=route_all.at[pl.ds(o, 1)],
                dst_ref=route_all.at[pl.ds(o, 1)],
                send_sem=r_send.at[dd - 1], recv_sem=r_recv.at[o],
                device_id=(s,), device_id_type=pl.DeviceIdType.MESH,
            ).wait_recv()

        eids = lax.broadcasted_iota(jnp.int32, (m, N_EXP), 1)
        route_own = route_ref[...]
        R = (route_own == eids)
        R_f = R.astype(jnp.float32)
        R_b = R.astype(jnp.bfloat16)

        offs = jnp.zeros((1, N_EXP), jnp.float32)
        for sp in range(N_DEV):
            R_sp = (route_all[sp] == eids).astype(jnp.float32)
            cnt = jnp.sum(R_sp, axis=0, keepdims=True)
            offs = offs + jnp.where(sp < s, cnt, 0.0)

        row = lax.broadcasted_iota(jnp.int32, (m, m), 0)
        col = lax.broadcasted_iota(jnp.int32, (m, m), 1)
        L = (col < row).astype(jnp.bfloat16)
        cum = jnp.dot(L, R_b, preferred_element_type=jnp.float32)
        rank = jnp.sum(cum * R_f, axis=1, keepdims=True)
        off_tok = jnp.sum(offs * R_f, axis=1, keepdims=True)
        keep = ((rank + off_tok) < float(CAP)).astype(jnp.bfloat16)

        xb_ref[...] = x_ref[...].astype(jnp.bfloat16)

        def expert_mm(e, init=False):
            msk = keep * (route_own == e).astype(jnp.bfloat16)
            w_e = w_all[pl.ds(e, 1)][0]
            mm = jnp.dot(xb_ref[...] * msk, w_e,
                         preferred_element_type=jnp.float32)
            out_ref[...] = mm if init else out_ref[...] + mm

        for j in range(E_LOC):
            expert_mm(s * E_LOC + j, init=(j == 0))

        for k in range(N_DEV - 1):
            rR[k].wait_recv()
            rL[k].wait_recv()
            if k + 1 < N_DEV - 1:
                sR[k + 1].start()
                sL[k + 1].start()
            oR = lax.rem(s - k - 1 + N_DEV, N_DEV)
            oL = lax.rem(s + k + 1, N_DEV)
            for j in range(HALF):
                expert_mm(oR * E_LOC + j)
            for j in range(HALF):
                expert_mm(oL * E_LOC + HALF + j)

        for k in range(N_DEV - 1):
            sR[k].wait_send()
            sL[k].wait_send()
        for rs in route_sends:
            rs.wait_send()

    return pl.pallas_call(
        body,
        out_shape=jax.ShapeDtypeStruct((m, h), jnp.float32),
        in_specs=[pl.BlockSpec(memory_space=pltpu.VMEM)] * 3,
        out_specs=pl.BlockSpec(memory_space=pltpu.VMEM),
        scratch_shapes=[
            pltpu.VMEM((N_EXP, d, h), jnp.bfloat16),
            pltpu.VMEM((N_DEV, m, 1), jnp.int32),
            pltpu.VMEM((m, d), jnp.bfloat16),
            pltpu.SemaphoreType.DMA((N_DEV - 1,)),
            pltpu.SemaphoreType.DMA((N_DEV - 1,)),
            pltpu.SemaphoreType.DMA((N_DEV - 1,)),
            pltpu.SemaphoreType.DMA((N_DEV - 1,)),
            pltpu.SemaphoreType.DMA((N_DEV - 1,)),
            pltpu.SemaphoreType.DMA((N_DEV,)),
        ],
        compiler_params=pltpu.CompilerParams(
            collective_id=0, vmem_limit_bytes=100 * 1024 * 1024
        ),
    )(x, route_idx, expert_W)
